# baseline (speedup 1.0000x reference)
import jax
import jax.numpy as jnp
from jax import lax
from jax.experimental import pallas as pl
from jax.experimental.pallas import tpu as pltpu
from jax.experimental.pallas import tpu_sc as plsc

_B = 16384
_SEQ = 50
_D = 32
_NCLS = 2
_V = 1_000_000
_NC = 2
_NS = 16
_NW = _NC * _NS
_RPW = _B // _NW
_CB = 16
_NCHUNK = _RPW // _CB
_GROWS = _CB * _SEQ

_VBLK = 7813
_VPAD = _VBLK * 128
_CONV_ROWS = _VPAD // 4
_GW = 512
_NGRP = 999936 // _GW
_CT = 62
_RW = 33


def _xpose(tb, ob, nw, scat):
    def chunk(k, _):
        w0 = k * 16
        base = scat + w0 * _RW
        for d in range(_D):
            v = tb[d, pl.ds(w0, 16)]
            plsc.store_scatter(ob, [base + d], v)
        return ()

    lax.fori_loop(0, nw // 16, chunk, ())


def _conv_body(t_t_ref, aux_ref, out_ref,
               tb0, tb1, ob0, ob1, si0, si1, so0, so1):
    wid = lax.axis_index("s") * _NC + lax.axis_index("c")
    scat = jnp.arange(16, dtype=jnp.int32) * _RW
    bufs = ((tb0, ob0, si0, so0), (tb1, ob1, si1, so1))

    def in_cp(g, tb, si):
        return pltpu.make_async_copy(
            t_t_ref.at[:, pl.ds(g * _GW, _GW)], tb, si
        )

    def out_cp(g, ob, so):
        return pltpu.make_async_copy(
            ob, out_ref.at[pl.ds(g * _GW * _RW, _GW * _RW)], so
        )

    for par in (0, 1):
        tb, ob, si, so = bufs[par]
        g = wid + _NW * par

        @pl.when(g < _NGRP)
        def _():
            in_cp(g, tb, si).start()

    def body(u, _):
        for par in (0, 1):
            tb, ob, si, so = bufs[par]
            t = 2 * u + par
            g = wid + _NW * t

            @pl.when(u > 0)
            def _():
                out_cp(jnp.maximum(g - 2 * _NW, 0), ob, so).wait()

            @pl.when(g < _NGRP)
            def _():
                in_cp(g, tb, si).wait()
                _xpose(tb, ob, _GW, scat)
                out_cp(g, ob, so).start()

                @pl.when(g + 2 * _NW < _NGRP)
                def _():
                    in_cp(g + 2 * _NW, tb, si).start()
        return ()

    lax.fori_loop(0, _CT // 2, body, ())

    out_cp(wid + _NW * (_CT - 2), ob0, so0).wait()

    @pl.when(wid == 0)
    def _():
        out_cp(_NW * (_CT - 1), ob1, so1).wait()
        pltpu.sync_copy(aux_ref, tb0.at[:, pl.ds(0, 128)])
        _xpose(tb0, ob0, 128, scat)
        pltpu.sync_copy(
            ob0.at[pl.ds(0, 64 * _RW)],
            out_ref.at[pl.ds(999936 * _RW, 64 * _RW)],
        )


def _convert_table(t_t, aux):
    mesh = plsc.VectorSubcoreMesh(core_axis_name="c", subcore_axis_name="s")
    f = pl.kernel(
        _conv_body,
        out_type=jax.ShapeDtypeStruct((_VPAD * _RW,), jnp.float32),
        mesh=mesh,
        scratch_types=[
            pltpu.VMEM((_D, _GW), jnp.float32),
            pltpu.VMEM((_D, _GW), jnp.float32),
            pltpu.VMEM((_GW * _RW,), jnp.float32),
            pltpu.VMEM((_GW * _RW,), jnp.float32),
            pltpu.SemaphoreType.DMA,
            pltpu.SemaphoreType.DMA,
            pltpu.SemaphoreType.DMA,
            pltpu.SemaphoreType.DMA,
        ],
        compiler_params=pltpu.CompilerParams(
            needs_layout_passes=False, use_tc_tiling_on_sc=True
        ),
    )
    return f(t_t, aux)


def _transpose_table(t_t):
    return pl.pallas_call(
        _tr_body,
        grid=(_TGRID,),
        in_specs=[pl.BlockSpec((_D, _TB), lambda k: (0, k))],
        out_specs=pl.BlockSpec((_TB // 4, 128), lambda k: (k, 0)),
        out_shape=jax.ShapeDtypeStruct((_V * _D // 128, 128), jnp.float32),
    )(t_t)


def _sc_body(x_ref, tab_ref, w_ref, out_ref,
             idx_v, idxc_v, emb_v, w_v, out_v, red_v, sem):
    wid = lax.axis_index("s") * _NC + lax.axis_index("c")
    pltpu.sync_copy(x_ref.at[:, pl.ds(wid * _RPW, _RPW)], idx_v)
    pltpu.sync_copy(w_ref, w_v)
    lanes = jnp.arange(16, dtype=jnp.int32)

    def do_chunk(c, _):
        for s in range(_SEQ):
            idxc_v[pl.ds(s * _CB, _CB)] = idx_v[s, pl.ds(c * _CB, _CB)]
        pltpu.async_copy(tab_ref.at[idxc_v], emb_v, sem).wait()

        def s_body(s, accs):
            a0, a1 = accs
            o = s * _D
            w0l = w_v[0, pl.ds(o, 16)]
            w0h = w_v[0, pl.ds(o + 16, 16)]
            w1l = w_v[1, pl.ds(o, 16)]
            w1h = w_v[1, pl.ds(o + 16, 16)]
            na0, na1 = [], []
            for r in range(_CB):
                el = emb_v[s * _CB + r, pl.ds(0, 16)]
                eh = emb_v[s * _CB + r, pl.ds(16, 16)]
                na0.append(a0[r] + el * w0l + eh * w0h)
                na1.append(a1[r] + el * w1l + eh * w1h)
            return na0, na1

        zero = jnp.zeros((16,), jnp.float32)
        a0, a1 = lax.fori_loop(0, _SEQ, s_body, ([zero] * _CB, [zero] * _CB))
        base = lanes * 16
        for cls, acc in ((0, a0), (1, a1)):
            for r in range(_CB):
                red_v[pl.ds(r * 16, 16)] = acc[r]
            tot = plsc.load_gather(red_v, [base])
            for j in range(1, 16):
                tot = tot + plsc.load_gather(red_v, [base + j])
            out_v[cls, pl.ds(c * _CB, _CB)] = tot
        return ()

    lax.fori_loop(0, _NCHUNK, do_chunk, ())
    pltpu.sync_copy(out_v, out_ref.at[wid])


@jax.jit
def _run(x_t, table, w):
    t_t = table.T
    aux = jnp.pad(t_t[:, _VBLK * 128 - 128:], ((0, 0), (0, 64)))
    tbl_lin = _convert_table(t_t, aux).reshape(_VPAD, _RW)
    mesh = plsc.VectorSubcoreMesh(core_axis_name="c", subcore_axis_name="s")
    f = pl.kernel(
        _sc_body,
        out_type=jax.ShapeDtypeStruct((_NW, _NCLS, _RPW), jnp.float32),
        mesh=mesh,
        scratch_types=[
            pltpu.VMEM((_SEQ, _RPW), jnp.int32),
            pltpu.VMEM((_GROWS,), jnp.int32),
            pltpu.VMEM((_GROWS, _RW), jnp.float32),
            pltpu.VMEM((_NCLS, _SEQ * _D), jnp.float32),
            pltpu.VMEM((_NCLS, _RPW), jnp.float32),
            pltpu.VMEM((256,), jnp.float32),
            pltpu.SemaphoreType.DMA,
        ],
        compiler_params=pltpu.CompilerParams(
            needs_layout_passes=False, use_tc_tiling_on_sc=False
        ),
    )
    return f(x_t, tbl_lin, w)


def kernel(x, table, W, b):
    out = _run(x.T.astype(jnp.int32), table, W)
    return out.transpose(0, 2, 1).reshape(_B, _NCLS) + b


# speedup vs baseline: 5.0785x; 5.0785x over previous
import jax
import jax.numpy as jnp
from jax import lax
from jax.experimental import pallas as pl
from jax.experimental.pallas import tpu as pltpu
from jax.experimental.pallas import tpu_sc as plsc

_B = 16384
_SEQ = 50
_D = 32
_NCLS = 2
_V = 1_000_000
_NC = 2
_NS = 16
_NW = _NC * _NS
_RPW = _B // _NW
_CB = 16
_NCHUNK = _RPW // _CB
_GROWS = _CB * _SEQ

_VBLK = 7813
_VPAD = _VBLK * 128
_CONV_ROWS = _VPAD // 4
_GW = 512
_NGRP = 999936 // _GW
_CT = 62


def _xpose(tb, ob, nw, iot):
    iot32 = iot * _D

    def chunk(k, _):
        w0 = k * 16
        wv = iot + w0
        base = iot32 + w0 * _D
        for r in range(16):
            dlo = (iot + r) & 15
            sc = base + dlo
            lo = plsc.load_gather(tb, [dlo, wv])
            hi = plsc.load_gather(tb, [dlo + 16, wv])
            plsc.store_scatter(ob, [sc], lo)
            plsc.store_scatter(ob, [sc + 16], hi)
        return ()

    lax.fori_loop(0, nw // 16, chunk, ())


def _conv_body(t_t_ref, aux_ref, out_ref,
               tb0, tb1, ob0, ob1, si0, si1, so0, so1):
    wid = lax.axis_index("s") * _NC + lax.axis_index("c")
    iot = jnp.arange(16, dtype=jnp.int32)
    bufs = ((tb0, ob0, si0, so0), (tb1, ob1, si1, so1))

    def in_cp(g, tb, si):
        return pltpu.make_async_copy(
            t_t_ref.at[:, pl.ds(g * _GW, _GW)], tb, si
        )

    def out_cp(g, ob, so):
        return pltpu.make_async_copy(
            ob, out_ref.at[pl.ds(g * _GW * _D, _GW * _D)], so
        )

    for par in (0, 1):
        tb, ob, si, so = bufs[par]
        g = wid + _NW * par

        @pl.when(g < _NGRP)
        def _():
            in_cp(g, tb, si).start()

    def body(u, _):
        for par in (0, 1):
            tb, ob, si, so = bufs[par]
            t = 2 * u + par
            g = wid + _NW * t

            @pl.when(u > 0)
            def _():
                out_cp(jnp.maximum(g - 2 * _NW, 0), ob, so).wait()

            @pl.when(g < _NGRP)
            def _():
                in_cp(g, tb, si).wait()
                _xpose(tb, ob, _GW, iot)
                out_cp(g, ob, so).start()

                @pl.when(g + 2 * _NW < _NGRP)
                def _():
                    in_cp(g + 2 * _NW, tb, si).start()
        return ()

    lax.fori_loop(0, _CT // 2, body, ())

    out_cp(wid + _NW * (_CT - 2), ob0, so0).wait()

    @pl.when(wid == 0)
    def _():
        out_cp(_NW * (_CT - 1), ob1, so1).wait()
        pltpu.sync_copy(aux_ref, tb0.at[:, pl.ds(0, 128)])
        _xpose(tb0, ob0, 128, iot)
        pltpu.sync_copy(
            ob0.at[pl.ds(0, 64 * _D)],
            out_ref.at[pl.ds(999936 * _D, 64 * _D)],
        )


def _convert_table(t_t, aux):
    mesh = plsc.VectorSubcoreMesh(core_axis_name="c", subcore_axis_name="s")
    f = pl.kernel(
        _conv_body,
        out_type=jax.ShapeDtypeStruct((_VPAD * _D,), jnp.float32),
        mesh=mesh,
        scratch_types=[
            pltpu.VMEM((_D, _GW), jnp.float32),
            pltpu.VMEM((_D, _GW), jnp.float32),
            pltpu.VMEM((_GW * _D,), jnp.float32),
            pltpu.VMEM((_GW * _D,), jnp.float32),
            pltpu.SemaphoreType.DMA,
            pltpu.SemaphoreType.DMA,
            pltpu.SemaphoreType.DMA,
            pltpu.SemaphoreType.DMA,
        ],
        compiler_params=pltpu.CompilerParams(
            needs_layout_passes=False, use_tc_tiling_on_sc=True
        ),
    )
    return f(t_t, aux)


def _sc_body(x_ref, tab_ref, w_ref, out_ref,
             idx_v, idxc_v, emb_v, w_v, out_v, red_v, sem):
    wid = lax.axis_index("s") * _NC + lax.axis_index("c")
    pltpu.sync_copy(x_ref.at[:, pl.ds(wid * _RPW, _RPW)], idx_v)
    pltpu.sync_copy(w_ref, w_v)
    lanes = jnp.arange(16, dtype=jnp.int32)

    def do_chunk(c, _):
        for s in range(_SEQ):
            idxc_v[pl.ds(s * _CB, _CB)] = idx_v[s, pl.ds(c * _CB, _CB)]
        pltpu.async_copy(tab_ref.at[idxc_v], emb_v, sem).wait()

        def s_body(s, accs):
            a0, a1 = accs
            o = s * _D
            w0l = w_v[0, pl.ds(o, 16)]
            w0h = w_v[0, pl.ds(o + 16, 16)]
            w1l = w_v[1, pl.ds(o, 16)]
            w1h = w_v[1, pl.ds(o + 16, 16)]
            na0, na1 = [], []
            for r in range(_CB):
                el = emb_v[s * _CB + r, pl.ds(0, 16)]
                eh = emb_v[s * _CB + r, pl.ds(16, 16)]
                na0.append(a0[r] + el * w0l + eh * w0h)
                na1.append(a1[r] + el * w1l + eh * w1h)
            return na0, na1

        zero = jnp.zeros((16,), jnp.float32)
        a0, a1 = lax.fori_loop(0, _SEQ, s_body, ([zero] * _CB, [zero] * _CB))
        base = lanes * 16
        for cls, acc in ((0, a0), (1, a1)):
            for r in range(_CB):
                red_v[pl.ds(r * 16, 16)] = acc[r]
            tot = plsc.load_gather(red_v, [base])
            for j in range(1, 16):
                tot = tot + plsc.load_gather(red_v, [base + j])
            out_v[cls, pl.ds(c * _CB, _CB)] = tot
        return ()

    lax.fori_loop(0, _NCHUNK, do_chunk, ())
    pltpu.sync_copy(out_v, out_ref.at[wid])


@jax.jit
def _run(x_t, table, w):
    t_t = table.T
    aux = jnp.pad(t_t[:, _VBLK * 128 - 128:], ((0, 0), (0, 64)))
    tbl_lin = _convert_table(t_t, aux).reshape(_VPAD, _D)
    mesh = plsc.VectorSubcoreMesh(core_axis_name="c", subcore_axis_name="s")
    f = pl.kernel(
        _sc_body,
        out_type=jax.ShapeDtypeStruct((_NW, _NCLS, _RPW), jnp.float32),
        mesh=mesh,
        scratch_types=[
            pltpu.VMEM((_SEQ, _RPW), jnp.int32),
            pltpu.VMEM((_GROWS,), jnp.int32),
            pltpu.VMEM((_GROWS, _D), jnp.float32),
            pltpu.VMEM((_NCLS, _SEQ * _D), jnp.float32),
            pltpu.VMEM((_NCLS, _RPW), jnp.float32),
            pltpu.VMEM((256,), jnp.float32),
            pltpu.SemaphoreType.DMA,
        ],
        compiler_params=pltpu.CompilerParams(
            needs_layout_passes=False, use_tc_tiling_on_sc=False
        ),
    )
    return f(x_t, tbl_lin, w)


def kernel(x, table, W, b):
    out = _run(x.T.astype(jnp.int32), table, W)
    return out.transpose(0, 2, 1).reshape(_B, _NCLS) + b


# speedup vs baseline: 5.8733x; 1.1565x over previous
import jax
import jax.numpy as jnp
from jax import lax
from jax.experimental import pallas as pl
from jax.experimental.pallas import tpu as pltpu
from jax.experimental.pallas import tpu_sc as plsc

_B = 16384
_SEQ = 50
_D = 32
_NCLS = 2
_V = 1_000_000
_NC = 2
_NS = 16
_NW = _NC * _NS
_RPW = _B // _NW
_CB = 16
_NCHUNK = _RPW // _CB
_GROWS = _CB * _SEQ

_VBLK = 7813
_VPAD = _VBLK * 128
_CONV_ROWS = _VPAD // 4
_GW = 512
_NGRP = 999936 // _GW
_CT = 62


def _xpose(tb, ob, nw, iot):
    iot32 = iot * _D

    def chunk(k, _):
        w0 = k * 16
        wv = iot + w0
        base = iot32 + w0 * _D
        for r in range(16):
            dlo = (iot + r) & 15
            sc = base + dlo
            lo = plsc.load_gather(tb, [dlo, wv])
            hi = plsc.load_gather(tb, [dlo + 16, wv])
            plsc.store_scatter(ob, [sc], lo)
            plsc.store_scatter(ob, [sc + 16], hi)
        return ()

    lax.fori_loop(0, nw // 16, chunk, ())


def _conv_body(t_t_ref, aux_ref, out_ref,
               tb0, tb1, ob0, ob1, si0, si1, so0, so1):
    wid = lax.axis_index("s") * _NC + lax.axis_index("c")
    iot = jnp.arange(16, dtype=jnp.int32)
    bufs = ((tb0, ob0, si0, so0), (tb1, ob1, si1, so1))

    def in_cp(g, tb, si):
        return pltpu.make_async_copy(
            t_t_ref.at[:, pl.ds(g * _GW, _GW)], tb, si
        )

    def out_cp(g, ob, so):
        return pltpu.make_async_copy(
            ob, out_ref.at[pl.ds(g * _GW * _D, _GW * _D)], so
        )

    for par in (0, 1):
        tb, ob, si, so = bufs[par]
        g = wid + _NW * par

        @pl.when(g < _NGRP)
        def _():
            in_cp(g, tb, si).start()

    def body(u, _):
        for par in (0, 1):
            tb, ob, si, so = bufs[par]
            t = 2 * u + par
            g = wid + _NW * t

            @pl.when(u > 0)
            def _():
                out_cp(jnp.maximum(g - 2 * _NW, 0), ob, so).wait()

            @pl.when(g < _NGRP)
            def _():
                in_cp(g, tb, si).wait()
                _xpose(tb, ob, _GW, iot)
                out_cp(g, ob, so).start()

                @pl.when(g + 2 * _NW < _NGRP)
                def _():
                    in_cp(g + 2 * _NW, tb, si).start()
        return ()

    lax.fori_loop(0, _CT // 2, body, ())

    out_cp(wid + _NW * (_CT - 2), ob0, so0).wait()

    @pl.when(wid == 0)
    def _():
        out_cp(_NW * (_CT - 1), ob1, so1).wait()
        pltpu.sync_copy(aux_ref, tb0.at[:, pl.ds(0, 128)])
        _xpose(tb0, ob0, 128, iot)
        pltpu.sync_copy(
            ob0.at[pl.ds(0, 64 * _D)],
            out_ref.at[pl.ds(999936 * _D, 64 * _D)],
        )


def _convert_table(t_t, aux):
    mesh = plsc.VectorSubcoreMesh(core_axis_name="c", subcore_axis_name="s")
    f = pl.kernel(
        _conv_body,
        out_type=jax.ShapeDtypeStruct((_VPAD * _D,), jnp.float32),
        mesh=mesh,
        scratch_types=[
            pltpu.VMEM((_D, _GW), jnp.float32),
            pltpu.VMEM((_D, _GW), jnp.float32),
            pltpu.VMEM((_GW * _D,), jnp.float32),
            pltpu.VMEM((_GW * _D,), jnp.float32),
            pltpu.SemaphoreType.DMA,
            pltpu.SemaphoreType.DMA,
            pltpu.SemaphoreType.DMA,
            pltpu.SemaphoreType.DMA,
        ],
        compiler_params=pltpu.CompilerParams(
            needs_layout_passes=False, use_tc_tiling_on_sc=True
        ),
    )
    return f(t_t, aux)


def _sc_body(x_ref, tab_ref, w_ref, out_ref,
             idx_v, idxc0, idxc1, emb0, emb1, w_v, out_v, red_v, sg0, sg1):
    wid = lax.axis_index("s") * _NC + lax.axis_index("c")
    pltpu.sync_copy(x_ref.at[:, pl.ds(wid * _RPW, _RPW)], idx_v)
    pltpu.sync_copy(w_ref, w_v)
    lanes = jnp.arange(16, dtype=jnp.int32)
    bufs = ((idxc0, emb0, sg0), (idxc1, emb1, sg1))

    def repack(c, idxc):
        for s in range(_SEQ):
            idxc[pl.ds(s * _CB, _CB)] = idx_v[s, pl.ds(c * _CB, _CB)]

    def gather(idxc, emb, sg):
        return pltpu.make_async_copy(tab_ref.at[idxc], emb, sg)

    def compute(c, emb_v):

        def s_body(s, accs):
            a0, a1 = accs
            o = s * _D
            w0l = w_v[0, pl.ds(o, 16)]
            w0h = w_v[0, pl.ds(o + 16, 16)]
            w1l = w_v[1, pl.ds(o, 16)]
            w1h = w_v[1, pl.ds(o + 16, 16)]
            na0, na1 = [], []
            for r in range(_CB):
                el = emb_v[s * _CB + r, pl.ds(0, 16)]
                eh = emb_v[s * _CB + r, pl.ds(16, 16)]
                na0.append(a0[r] + el * w0l + eh * w0h)
                na1.append(a1[r] + el * w1l + eh * w1h)
            return na0, na1

        zero = jnp.zeros((16,), jnp.float32)
        a0, a1 = lax.fori_loop(0, _SEQ, s_body, ([zero] * _CB, [zero] * _CB))
        base = lanes * 16
        for cls, acc in ((0, a0), (1, a1)):
            for r in range(_CB):
                red_v[pl.ds(r * 16, 16)] = acc[r]
            tot = plsc.load_gather(red_v, [base])
            for j in range(1, 16):
                tot = tot + plsc.load_gather(red_v, [base + j])
            out_v[cls, pl.ds(c * _CB, _CB)] = tot

    repack(0, idxc0)
    gather(idxc0, emb0, sg0).start()

    def pair(u, _):
        for par in (0, 1):
            idxc, emb, sg = bufs[par]
            idxn, embn, sgn = bufs[1 - par]
            c = 2 * u + par
            gather(idxc, emb, sg).wait()

            @pl.when(c < _NCHUNK - 1)
            def _():
                repack(c + 1, idxn)
                gather(idxn, embn, sgn).start()

            compute(c, emb)
        return ()

    lax.fori_loop(0, _NCHUNK // 2, pair, ())
    pltpu.sync_copy(out_v, out_ref.at[wid])


@jax.jit
def _run(x_t, table, w):
    t_t = table.T
    aux = jnp.pad(t_t[:, _VBLK * 128 - 128:], ((0, 0), (0, 64)))
    tbl_lin = _convert_table(t_t, aux).reshape(_VPAD, _D)
    mesh = plsc.VectorSubcoreMesh(core_axis_name="c", subcore_axis_name="s")
    f = pl.kernel(
        _sc_body,
        out_type=jax.ShapeDtypeStruct((_NW, _NCLS, _RPW), jnp.float32),
        mesh=mesh,
        scratch_types=[
            pltpu.VMEM((_SEQ, _RPW), jnp.int32),
            pltpu.VMEM((_GROWS,), jnp.int32),
            pltpu.VMEM((_GROWS,), jnp.int32),
            pltpu.VMEM((_GROWS, _D), jnp.float32),
            pltpu.VMEM((_GROWS, _D), jnp.float32),
            pltpu.VMEM((_NCLS, _SEQ * _D), jnp.float32),
            pltpu.VMEM((_NCLS, _RPW), jnp.float32),
            pltpu.VMEM((256,), jnp.float32),
            pltpu.SemaphoreType.DMA,
            pltpu.SemaphoreType.DMA,
        ],
        compiler_params=pltpu.CompilerParams(
            needs_layout_passes=False, use_tc_tiling_on_sc=False
        ),
    )
    return f(x_t, tbl_lin, w)


def kernel(x, table, W, b):
    out = _run(x.T.astype(jnp.int32), table, W)
    return out.transpose(0, 2, 1).reshape(_B, _NCLS) + b
